# baseline (speedup 1.0000x reference)
import functools

import jax
import jax.numpy as jnp
from jax import lax
from jax.experimental import pallas as pl
from jax.experimental.pallas import tpu as pltpu
from jax.experimental.pallas import tpu_sc as plsc

D_FEAT = 128
D_AUG = 144
CHUNK = 128
NW = 32


def _sc_scatter_partials(src, dst, h_aug, n_pad, rps):
    e_pad = src.shape[0] - 4 * CHUNK
    cpw = e_pad // CHUNK // NW
    mesh = plsc.VectorSubcoreMesh(core_axis_name="c", subcore_axis_name="s")

    @functools.partial(
        pl.kernel,
        out_type=jax.ShapeDtypeStruct((2, n_pad, D_AUG), jnp.float32),
        mesh=mesh,
        compiler_params=pltpu.CompilerParams(use_tc_tiling_on_sc=False),
        scratch_types=[
            *([pltpu.VMEM((CHUNK,), jnp.int32)] * 16),
            pltpu.VMEM((CHUNK, D_AUG), jnp.float32),
            pltpu.VMEM((CHUNK, D_AUG), jnp.float32),
            pltpu.VMEM((CHUNK, D_AUG), jnp.float32),
            pltpu.VMEM((CHUNK, D_AUG), jnp.float32),
            pltpu.VMEM((8, D_AUG), jnp.float32),
            pltpu.VMEM_SHARED((n_pad, D_AUG), jnp.float32),
            pltpu.SemaphoreType.DMA,
        ],
    )
    def body(src_hbm, dst_hbm, h_hbm, out_hbm,
             sa0, sa1, sa2, sa3, sb0, sb1, sb2, sb3,
             da0, da1, da2, da3, db0, db1, db2, db3,
             r0, r1, r2, r3, zbuf, acc, sem):
        c = lax.axis_index("c")
        s = lax.axis_index("s")
        zeros16 = jnp.zeros((16,), jnp.float32)
        for r in range(8):
            for j in range(D_AUG // 16):
                zbuf[r, pl.ds(j * 16, 16)] = zeros16

        def zero_row(i, carry):
            off = pl.multiple_of(s * rps + i * 8, 8)
            pltpu.sync_copy(zbuf, acc.at[pl.ds(off, 8)])
            return carry

        lax.fori_loop(0, rps // 8, zero_row, 0)
        plsc.subcore_barrier()

        wid = s * 2 + c
        base = wid * cpw
        SA = (sa0, sa1, sa2, sa3)
        SB = (sb0, sb1, sb2, sb3)
        DA = (da0, da1, da2, da3)
        DB = (db0, db1, db2, db3)
        rbufs = (r0, r1, r2, r3)
        ngroups = cpw // 4

        def s_src(cidx):
            return src_hbm.at[pl.ds((base + cidx) * CHUNK, CHUNK)]

        def d_src(cidx):
            return dst_hbm.at[pl.ds((base + cidx) * CHUNK, CHUNK)]

        for j in range(4):
            pltpu.sync_copy(s_src(j), SA[j])
            pltpu.sync_copy(d_src(j), DA[j])

        def group_pair(i, carry):
            for cur_s, cur_d, nxt_s, nxt_d, gofs in (
                    (SA, DA, SB, DB, 0), (SB, DB, SA, DA, 1)):
                g = i * 2 + gofs
                nb = (g + 1) * 4
                handles = []
                for j in range(4):
                    handles.append(pltpu.async_copy(s_src(nb + j), nxt_s[j], sem))
                    handles.append(pltpu.async_copy(d_src(nb + j), nxt_d[j], sem))
                for j in range(4):
                    handles.append(
                        pltpu.async_copy(h_hbm.at[cur_s[j]], rbufs[j], sem))
                for h in handles:
                    h.wait()
                for j in range(4):
                    pltpu.sync_copy(rbufs[j], acc.at[cur_d[j]], add=True)
            return carry

        lax.fori_loop(0, ngroups // 2, group_pair, 0)
        plsc.subcore_barrier()

        roff = pl.multiple_of(s * rps, 8)
        pltpu.sync_copy(acc.at[pl.ds(roff, rps)], out_hbm.at[c, pl.ds(roff, rps)])

    return body(src, dst, h_aug)


def _tc_layer(partials, h_aug, w_l, w_r, b, n_tgt):
    def body(p_ref, h_ref, wl_ref, wr_ref, b_ref, o_ref):
        agg = p_ref[0, :n_tgt, :] + p_ref[1, :n_tgt, :]
        ssum = agg[:, :D_FEAT]
        cnt = agg[:, D_FEAT:D_FEAT + 1]
        mean = ssum / jnp.maximum(cnt, 1.0)
        x_tgt = h_ref[:n_tgt, :D_FEAT]
        out = (jnp.dot(mean, wl_ref[...], preferred_element_type=jnp.float32)
               + jnp.dot(x_tgt, wr_ref[...], preferred_element_type=jnp.float32)
               + b_ref[...])
        col = lax.broadcasted_iota(jnp.int32, (n_tgt, D_AUG - D_FEAT), 1)
        tail = jnp.where(col == 0, 1.0, 0.0).astype(jnp.float32)
        o_ref[...] = jnp.concatenate([out, tail], axis=1)

    return pl.pallas_call(
        body,
        out_shape=jax.ShapeDtypeStruct((n_tgt, D_AUG), jnp.float32),
    )(partials, h_aug, w_l, w_r, b)


def _tc_final(partials, h_aug, w_l, w_r, b, n_tgt, d_out):
    def body(p_ref, h_ref, wl_ref, wr_ref, b_ref, o_ref):
        agg = p_ref[0, :n_tgt, :] + p_ref[1, :n_tgt, :]
        ssum = agg[:, :D_FEAT]
        cnt = agg[:, D_FEAT:D_FEAT + 1]
        mean = ssum / jnp.maximum(cnt, 1.0)
        x_tgt = h_ref[:n_tgt, :D_FEAT]
        logits = (jnp.dot(mean, wl_ref[...], preferred_element_type=jnp.float32)
                  + jnp.dot(x_tgt, wr_ref[...], preferred_element_type=jnp.float32)
                  + b_ref[...])
        col = lax.broadcasted_iota(jnp.int32, logits.shape, 1)
        masked = jnp.where(col < d_out, logits, -1e30)
        m = jnp.max(masked, axis=1, keepdims=True)
        lse = jnp.log(jnp.sum(jnp.exp(masked - m), axis=1, keepdims=True))
        o_ref[...] = logits - m - lse

    return pl.pallas_call(
        body,
        out_shape=jax.ShapeDtypeStruct((n_tgt, D_FEAT), jnp.float32),
    )(partials, h_aug, w_l, w_r, b)


def _pad_edges(ei, e_pad, dst_pad):
    src = ei[0].astype(jnp.int32)
    dst = ei[1].astype(jnp.int32)
    extra = e_pad + 4 * CHUNK - src.shape[0]
    src = jnp.concatenate([src, jnp.zeros((extra,), jnp.int32)])
    dst = jnp.concatenate([dst, jnp.full((extra,), dst_pad, jnp.int32)])
    return src, dst


def _augment(h):
    n = h.shape[0]
    return jnp.concatenate(
        [h, jnp.ones((n, 1), jnp.float32), jnp.zeros((n, D_AUG - D_FEAT - 1), jnp.float32)],
        axis=1)


def kernel(x, edge_index0, edge_index1, edge_index2,
           W_l0, W_r0, b0, W_l1, W_r1, b1, W_l2, W_r2, b2):
    src0, dst0 = _pad_edges(edge_index0, 327680, 5000)
    src1, dst1 = _pad_edges(edge_index1, 163840, 2000)
    src2, dst2 = _pad_edges(edge_index2, 65536, 1000)

    h0 = _augment(x[:5000])

    p0 = _sc_scatter_partials(src0, dst0, h0, 5120, 320)
    h1 = _tc_layer(p0, h0, W_l0, W_r0, b0.reshape(1, D_FEAT), 5000)

    p1 = _sc_scatter_partials(src1, dst1, h1, 2048, 128)
    h2 = _tc_layer(p1, h1, W_l1, W_r1, b1.reshape(1, D_FEAT), 2000)

    p2 = _sc_scatter_partials(src2, dst2, h2, 1024, 64)
    d_out = W_l2.shape[1]
    wl2 = jnp.zeros((D_FEAT, D_FEAT), jnp.float32).at[:, :d_out].set(W_l2)
    wr2 = jnp.zeros((D_FEAT, D_FEAT), jnp.float32).at[:, :d_out].set(W_r2)
    b2p = jnp.zeros((1, D_FEAT), jnp.float32).at[0, :d_out].set(b2)
    out = _tc_final(p2, h2, wl2, wr2, b2p, 1000, d_out)
    return out[:, :d_out]


# speedup vs baseline: 1.0015x; 1.0015x over previous
import functools

import jax
import jax.numpy as jnp
from jax import lax
from jax.experimental import pallas as pl
from jax.experimental.pallas import tpu as pltpu
from jax.experimental.pallas import tpu_sc as plsc

D_FEAT = 128
D_AUG = 144
CHUNK = 128
NW = 32


def _sc_scatter_partials(src, dst, h_aug, n_pad, rps):
    e_pad = src.shape[0] - 4 * CHUNK
    cpw = e_pad // CHUNK // NW
    mesh = plsc.VectorSubcoreMesh(core_axis_name="c", subcore_axis_name="s")

    @functools.partial(
        pl.kernel,
        out_type=jax.ShapeDtypeStruct((2, n_pad, D_AUG), jnp.float32),
        mesh=mesh,
        compiler_params=pltpu.CompilerParams(use_tc_tiling_on_sc=False),
        scratch_types=[
            *([pltpu.VMEM((CHUNK,), jnp.int32)] * 16),
            pltpu.VMEM((CHUNK, D_AUG), jnp.float32),
            pltpu.VMEM((CHUNK, D_AUG), jnp.float32),
            pltpu.VMEM((CHUNK, D_AUG), jnp.float32),
            pltpu.VMEM((CHUNK, D_AUG), jnp.float32),
            pltpu.VMEM((8, D_AUG), jnp.float32),
            pltpu.VMEM_SHARED((n_pad, D_AUG), jnp.float32),
            pltpu.SemaphoreType.DMA,
        ],
    )
    def body(src_hbm, dst_hbm, h_hbm, out_hbm,
             sa0, sa1, sa2, sa3, sb0, sb1, sb2, sb3,
             da0, da1, da2, da3, db0, db1, db2, db3,
             r0, r1, r2, r3, zbuf, acc, sem):
        c = lax.axis_index("c")
        s = lax.axis_index("s")
        zeros16 = jnp.zeros((16,), jnp.float32)
        for r in range(8):
            for j in range(D_AUG // 16):
                zbuf[r, pl.ds(j * 16, 16)] = zeros16

        def zero_row(i, carry):
            off = pl.multiple_of(s * rps + i * 8, 8)
            pltpu.sync_copy(zbuf, acc.at[pl.ds(off, 8)])
            return carry

        lax.fori_loop(0, rps // 8, zero_row, 0)
        plsc.subcore_barrier()

        wid = s * 2 + c
        base = wid * cpw
        SA = (sa0, sa1, sa2, sa3)
        SB = (sb0, sb1, sb2, sb3)
        DA = (da0, da1, da2, da3)
        DB = (db0, db1, db2, db3)
        rbufs = (r0, r1, r2, r3)
        ngroups = cpw // 4

        def s_src(cidx):
            return src_hbm.at[pl.ds((base + cidx) * CHUNK, CHUNK)]

        def d_src(cidx):
            return dst_hbm.at[pl.ds((base + cidx) * CHUNK, CHUNK)]

        for j in range(4):
            pltpu.sync_copy(s_src(j), SA[j])
            pltpu.sync_copy(d_src(j), DA[j])

        def group_pair(i, carry):
            for cur_s, cur_d, nxt_s, nxt_d, gofs in (
                    (SA, DA, SB, DB, 0), (SB, DB, SA, DA, 1)):
                g = i * 2 + gofs
                nb = (g + 1) * 4
                handles = []
                for j in range(4):
                    handles.append(pltpu.async_copy(s_src(nb + j), nxt_s[j], sem))
                    handles.append(pltpu.async_copy(d_src(nb + j), nxt_d[j], sem))
                for j in range(4):
                    handles.append(
                        pltpu.async_copy(h_hbm.at[cur_s[j]], rbufs[j], sem))
                for h in handles:
                    h.wait()
                for j in range(4):
                    pltpu.sync_copy(rbufs[j], acc.at[cur_d[j]], add=True)
            return carry

        lax.fori_loop(0, ngroups // 2, group_pair, 0)
        plsc.subcore_barrier()

        roff = pl.multiple_of(s * rps, 8)
        pltpu.sync_copy(acc.at[pl.ds(roff, rps)], out_hbm.at[c, pl.ds(roff, rps)])

    return body(src, dst, h_aug)


def _tc_layer(partials, h_aug, w_l, w_r, b, n_tgt):
    def body(p_ref, h_ref, wl_ref, wr_ref, b_ref, o_ref):
        agg = p_ref[0, :n_tgt, :] + p_ref[1, :n_tgt, :]
        ssum = agg[:, :D_FEAT]
        cnt = agg[:, D_FEAT:D_FEAT + 1]
        mean = ssum / jnp.maximum(cnt, 1.0)
        x_tgt = h_ref[:n_tgt, :D_FEAT]
        out = (jnp.dot(mean, wl_ref[...], preferred_element_type=jnp.float32)
               + jnp.dot(x_tgt, wr_ref[...], preferred_element_type=jnp.float32)
               + b_ref[...])
        col = lax.broadcasted_iota(jnp.int32, (n_tgt, D_AUG - D_FEAT), 1)
        tail = jnp.where(col == 0, 1.0, 0.0).astype(jnp.float32)
        o_ref[...] = jnp.concatenate([out, tail], axis=1)

    return pl.pallas_call(
        body,
        out_shape=jax.ShapeDtypeStruct((n_tgt, D_AUG), jnp.float32),
    )(partials, h_aug, w_l, w_r, b)


def _tc_final(partials, h_aug, w_l, w_r, b, n_tgt, d_out):
    def body(p_ref, h_ref, wl_ref, wr_ref, b_ref, o_ref):
        agg = p_ref[0, :n_tgt, :] + p_ref[1, :n_tgt, :]
        ssum = agg[:, :D_FEAT]
        cnt = agg[:, D_FEAT:D_FEAT + 1]
        mean = ssum / jnp.maximum(cnt, 1.0)
        x_tgt = h_ref[:n_tgt, :D_FEAT]
        logits = (jnp.dot(mean, wl_ref[...], preferred_element_type=jnp.float32)
                  + jnp.dot(x_tgt, wr_ref[...], preferred_element_type=jnp.float32)
                  + b_ref[...])
        col = lax.broadcasted_iota(jnp.int32, logits.shape, 1)
        masked = jnp.where(col < d_out, logits, -1e30)
        m = jnp.max(masked, axis=1, keepdims=True)
        lse = jnp.log(jnp.sum(jnp.exp(masked - m), axis=1, keepdims=True))
        o_ref[...] = logits - m - lse

    return pl.pallas_call(
        body,
        out_shape=jax.ShapeDtypeStruct((n_tgt, D_FEAT), jnp.float32),
    )(partials, h_aug, w_l, w_r, b)


def _pad_edges(ei, e_pad, dst_pad, n_pad):
    src = ei[0].astype(jnp.int32)
    dst = ei[1].astype(jnp.int32)
    extra = e_pad + 4 * CHUNK - src.shape[0]
    src = jnp.concatenate([src, jnp.zeros((extra,), jnp.int32)])
    fake = dst_pad + jnp.arange(extra, dtype=jnp.int32) % (n_pad - dst_pad)
    dst = jnp.concatenate([dst, fake])
    return src, dst


def _augment(h):
    n = h.shape[0]
    return jnp.concatenate(
        [h, jnp.ones((n, 1), jnp.float32), jnp.zeros((n, D_AUG - D_FEAT - 1), jnp.float32)],
        axis=1)


def kernel(x, edge_index0, edge_index1, edge_index2,
           W_l0, W_r0, b0, W_l1, W_r1, b1, W_l2, W_r2, b2):
    src0, dst0 = _pad_edges(edge_index0, 327680, 5000, 5120)
    src1, dst1 = _pad_edges(edge_index1, 163840, 2000, 2048)
    src2, dst2 = _pad_edges(edge_index2, 65536, 1000, 1024)

    h0 = _augment(x[:5000])

    p0 = _sc_scatter_partials(src0, dst0, h0, 5120, 320)
    h1 = _tc_layer(p0, h0, W_l0, W_r0, b0.reshape(1, D_FEAT), 5000)

    p1 = _sc_scatter_partials(src1, dst1, h1, 2048, 128)
    h2 = _tc_layer(p1, h1, W_l1, W_r1, b1.reshape(1, D_FEAT), 2000)

    p2 = _sc_scatter_partials(src2, dst2, h2, 1024, 64)
    d_out = W_l2.shape[1]
    wl2 = jnp.zeros((D_FEAT, D_FEAT), jnp.float32).at[:, :d_out].set(W_l2)
    wr2 = jnp.zeros((D_FEAT, D_FEAT), jnp.float32).at[:, :d_out].set(W_r2)
    b2p = jnp.zeros((1, D_FEAT), jnp.float32).at[0, :d_out].set(b2)
    out = _tc_final(p2, h2, wl2, wr2, b2p, 1000, d_out)
    return out[:, :d_out]


# speedup vs baseline: 1.1442x; 1.1425x over previous
import functools

import jax
import jax.numpy as jnp
from jax import lax
from jax.experimental import pallas as pl
from jax.experimental.pallas import tpu as pltpu
from jax.experimental.pallas import tpu_sc as plsc

D_FEAT = 128
D_AUG = 144
CHUNK = 128
NW = 32


def _sc_scatter_partials(src, dst, h_aug, n_pad, rps, cpw0, cpw1):
    mesh = plsc.VectorSubcoreMesh(core_axis_name="c", subcore_axis_name="s")

    @functools.partial(
        pl.kernel,
        out_type=jax.ShapeDtypeStruct((2, n_pad, D_AUG), jnp.float32),
        mesh=mesh,
        compiler_params=pltpu.CompilerParams(use_tc_tiling_on_sc=False),
        scratch_types=[
            *([pltpu.VMEM((CHUNK,), jnp.int32)] * 16),
            pltpu.VMEM((CHUNK, D_AUG), jnp.float32),
            pltpu.VMEM((CHUNK, D_AUG), jnp.float32),
            pltpu.VMEM((CHUNK, D_AUG), jnp.float32),
            pltpu.VMEM((CHUNK, D_AUG), jnp.float32),
            pltpu.VMEM((8, D_AUG), jnp.float32),
            pltpu.VMEM_SHARED((n_pad, D_AUG), jnp.float32),
            pltpu.SemaphoreType.DMA,
        ],
    )
    def body(src_hbm, dst_hbm, h_hbm, out_hbm,
             sa0, sa1, sa2, sa3, sb0, sb1, sb2, sb3,
             da0, da1, da2, da3, db0, db1, db2, db3,
             r0, r1, r2, r3, zbuf, acc, sem):
        c = lax.axis_index("c")
        s = lax.axis_index("s")
        zeros16 = jnp.zeros((16,), jnp.float32)
        for r in range(8):
            for j in range(D_AUG // 16):
                zbuf[r, pl.ds(j * 16, 16)] = zeros16

        def zero_row(i, carry):
            off = pl.multiple_of(s * rps + i * 8, 8)
            pltpu.sync_copy(zbuf, acc.at[pl.ds(off, 8)])
            return carry

        lax.fori_loop(0, rps // 8, zero_row, 0)
        plsc.subcore_barrier()

        base = jnp.where(c == 0, s * cpw0, 16 * cpw0 + s * cpw1)
        npairs = jnp.where(c == 0, cpw0 // 8, cpw1 // 8)
        SA = (sa0, sa1, sa2, sa3)
        SB = (sb0, sb1, sb2, sb3)
        DA = (da0, da1, da2, da3)
        DB = (db0, db1, db2, db3)
        rbufs = (r0, r1, r2, r3)

        def s_src(cidx):
            return src_hbm.at[pl.ds((base + cidx) * CHUNK, CHUNK)]

        def d_src(cidx):
            return dst_hbm.at[pl.ds((base + cidx) * CHUNK, CHUNK)]

        for j in range(4):
            pltpu.sync_copy(s_src(j), SA[j])
            pltpu.sync_copy(d_src(j), DA[j])

        def group_pair(i, carry):
            for cur_s, cur_d, nxt_s, nxt_d, gofs in (
                    (SA, DA, SB, DB, 0), (SB, DB, SA, DA, 1)):
                g = i * 2 + gofs
                nb = (g + 1) * 4
                handles = []
                for j in range(4):
                    handles.append(pltpu.async_copy(s_src(nb + j), nxt_s[j], sem))
                    handles.append(pltpu.async_copy(d_src(nb + j), nxt_d[j], sem))
                for j in range(4):
                    handles.append(
                        pltpu.async_copy(h_hbm.at[cur_s[j]], rbufs[j], sem))
                for h in handles:
                    h.wait()
                for j in range(4):
                    pltpu.sync_copy(rbufs[j], acc.at[cur_d[j]], add=True)
            return carry

        lax.fori_loop(0, npairs, group_pair, 0)
        plsc.subcore_barrier()

        roff = pl.multiple_of(s * rps, 8)
        pltpu.sync_copy(acc.at[pl.ds(roff, rps)], out_hbm.at[c, pl.ds(roff, rps)])

    return body(src, dst, h_aug)


def _tc_layer(partials, h_aug, w_l, w_r, b, n_tgt):
    def body(p_ref, h_ref, wl_ref, wr_ref, b_ref, o_ref):
        agg = p_ref[0, :n_tgt, :] + p_ref[1, :n_tgt, :]
        ssum = agg[:, :D_FEAT]
        cnt = agg[:, D_FEAT:D_FEAT + 1]
        mean = ssum / jnp.maximum(cnt, 1.0)
        x_tgt = h_ref[:n_tgt, :D_FEAT]
        out = (jnp.dot(mean, wl_ref[...], preferred_element_type=jnp.float32)
               + jnp.dot(x_tgt, wr_ref[...], preferred_element_type=jnp.float32)
               + b_ref[...])
        col = lax.broadcasted_iota(jnp.int32, (n_tgt, D_AUG - D_FEAT), 1)
        tail = jnp.where(col == 0, 1.0, 0.0).astype(jnp.float32)
        o_ref[...] = jnp.concatenate([out, tail], axis=1)

    return pl.pallas_call(
        body,
        out_shape=jax.ShapeDtypeStruct((n_tgt, D_AUG), jnp.float32),
    )(partials, h_aug, w_l, w_r, b)


def _tc_final(partials, h_aug, w_l, w_r, b, n_tgt, d_out):
    def body(p_ref, h_ref, wl_ref, wr_ref, b_ref, o_ref):
        agg = p_ref[0, :n_tgt, :] + p_ref[1, :n_tgt, :]
        ssum = agg[:, :D_FEAT]
        cnt = agg[:, D_FEAT:D_FEAT + 1]
        mean = ssum / jnp.maximum(cnt, 1.0)
        x_tgt = h_ref[:n_tgt, :D_FEAT]
        logits = (jnp.dot(mean, wl_ref[...], preferred_element_type=jnp.float32)
                  + jnp.dot(x_tgt, wr_ref[...], preferred_element_type=jnp.float32)
                  + b_ref[...])
        col = lax.broadcasted_iota(jnp.int32, logits.shape, 1)
        masked = jnp.where(col < d_out, logits, -1e30)
        m = jnp.max(masked, axis=1, keepdims=True)
        lse = jnp.log(jnp.sum(jnp.exp(masked - m), axis=1, keepdims=True))
        o_ref[...] = logits - m - lse

    return pl.pallas_call(
        body,
        out_shape=jax.ShapeDtypeStruct((n_tgt, D_FEAT), jnp.float32),
    )(partials, h_aug, w_l, w_r, b)


def _pad_edges(ei, e_pad, dst_pad, n_pad):
    src = ei[0].astype(jnp.int32)
    dst = ei[1].astype(jnp.int32)
    extra = e_pad + 4 * CHUNK - src.shape[0]
    src = jnp.concatenate([src, jnp.zeros((extra,), jnp.int32)])
    fake = dst_pad + jnp.arange(extra, dtype=jnp.int32) % (n_pad - dst_pad)
    dst = jnp.concatenate([dst, fake])
    return src, dst


def _augment(h):
    n = h.shape[0]
    return jnp.concatenate(
        [h, jnp.ones((n, 1), jnp.float32), jnp.zeros((n, D_AUG - D_FEAT - 1), jnp.float32)],
        axis=1)


def kernel(x, edge_index0, edge_index1, edge_index2,
           W_l0, W_r0, b0, W_l1, W_r1, b1, W_l2, W_r2, b2):
    src0, dst0 = _pad_edges(edge_index0, 327680, 5000, 5120)
    src1, dst1 = _pad_edges(edge_index1, 163840, 2000, 2048)
    src2, dst2 = _pad_edges(edge_index2, 65536, 1000, 1024)

    h0 = _augment(x[:5000])

    p0 = _sc_scatter_partials(src0, dst0, h0, 5120, 320, 120, 40)
    h1 = _tc_layer(p0, h0, W_l0, W_r0, b0.reshape(1, D_FEAT), 5000)

    p1 = _sc_scatter_partials(src1, dst1, h1, 2048, 128, 64, 16)
    h2 = _tc_layer(p1, h1, W_l1, W_r1, b1.reshape(1, D_FEAT), 2000)

    p2 = _sc_scatter_partials(src2, dst2, h2, 1024, 64, 24, 8)
    d_out = W_l2.shape[1]
    wl2 = jnp.zeros((D_FEAT, D_FEAT), jnp.float32).at[:, :d_out].set(W_l2)
    wr2 = jnp.zeros((D_FEAT, D_FEAT), jnp.float32).at[:, :d_out].set(W_r2)
    b2p = jnp.zeros((1, D_FEAT), jnp.float32).at[0, :d_out].set(b2)
    out = _tc_final(p2, h2, wl2, wr2, b2p, 1000, d_out)
    return out[:, :d_out]
